# baseline (speedup 1.0000x reference)
import functools

import jax
import jax.numpy as jnp
from jax import lax
from jax.experimental import pallas as pl
from jax.experimental.pallas import tpu as pltpu
from jax.experimental.pallas import tpu_sc as plsc

N_NODES = 10000
N_EDGES = 160000
CW = 64
PAIR = 128 // CW
EC = 128

NC = 2
NS = 16
NW = NC * NS

N_PAD = 10240
ROWS_PER_TILE = N_PAD // NS
PIECES = ROWS_PER_TILE // EC
E_PAD = 163840
EDGES_PER_W = E_PAD // NW
CHUNKS_PER_W = EDGES_PER_W // EC


def _sc_agg_body(C, with_deg, *refs):
    tab3 = refs[0]
    src_hbm = refs[1]
    dst_hbm = refs[2]
    pos = 3
    agg_out = refs[pos]; pos += 1
    if with_deg:
        deg_out = refs[pos]; pos += 1
    src_v = refs[pos]; pos += 1
    dst_v = refs[pos]; pos += 1
    gbuf0 = refs[pos]; pos += 1
    gbuf1 = refs[pos]; pos += 1
    zbuf = refs[pos]; pos += 1
    acc_sp = refs[pos]; pos += 1
    if with_deg:
        degbuf = refs[pos]; pos += 1
        ones_v = refs[pos]; pos += 1
        deg_sp = refs[pos]; pos += 1
    gsem0 = refs[pos]; pos += 1
    gsem1 = refs[pos]; pos += 1

    core = lax.axis_index("c")
    sub = lax.axis_index("s")
    wid = sub * NC + core
    ebase = wid * EDGES_PER_W
    rbase = sub * ROWS_PER_TILE

    pltpu.sync_copy(src_hbm.at[pl.ds(ebase, EDGES_PER_W)], src_v)
    pltpu.sync_copy(dst_hbm.at[pl.ds(wid * CHUNKS_PER_W, CHUNKS_PER_W)], dst_v)

    z16 = jnp.zeros((16,), jnp.float32)

    def zrow(r, carry):
        for k in range(CW // 16):
            zbuf[r, pl.ds(k * 16, 16)] = z16
        return carry

    lax.fori_loop(0, ROWS_PER_TILE, zrow, 0)
    if with_deg:
        one16 = jnp.ones((16,), jnp.float32)

        def dz(i, carry):
            degbuf[pl.ds(i * 16, 16)] = z16
            return carry

        lax.fori_loop(0, ROWS_PER_TILE // 16, dz, 0)
        for k in range(EC // 16):
            ones_v[pl.ds(k * 16, 16)] = one16

    for c in range(C):
        pltpu.sync_copy(zbuf, acc_sp.at[pl.ds(rbase, ROWS_PER_TILE)])
        if with_deg and c == 0:
            pltpu.sync_copy(degbuf, deg_sp.at[pl.ds(rbase, ROWS_PER_TILE)])
        plsc.subcore_barrier()

        tab = tab3.at[c]

        def start_gather(j, buf, sem):
            pltpu.async_copy(tab.at[src_v.at[pl.ds(j * EC, EC)]], buf, sem)

        def wait_gather(buf, sem):
            pltpu.make_async_copy(tab.at[pl.ds(0, EC)], buf, sem).wait()

        def scatter(j, buf):
            pltpu.sync_copy(buf, acc_sp.at[dst_v.at[j]], add=True)
            if with_deg and c == 0:
                pltpu.sync_copy(ones_v, deg_sp.at[dst_v.at[j]], add=True)

        start_gather(0, gbuf0, gsem0)

        def pair(k, carry):
            j0 = 2 * k
            start_gather(j0 + 1, gbuf1, gsem1)
            wait_gather(gbuf0, gsem0)
            scatter(j0, gbuf0)
            start_gather(j0 + 2, gbuf0, gsem0)
            wait_gather(gbuf1, gsem1)
            scatter(j0 + 1, gbuf1)
            return carry

        lax.fori_loop(0, CHUNKS_PER_W // 2 - 1, pair, 0)
        start_gather(CHUNKS_PER_W - 1, gbuf1, gsem1)
        wait_gather(gbuf0, gsem0)
        scatter(CHUNKS_PER_W - 2, gbuf0)
        wait_gather(gbuf1, gsem1)
        scatter(CHUNKS_PER_W - 1, gbuf1)
        plsc.subcore_barrier()

        pltpu.sync_copy(
            acc_sp.at[pl.ds(rbase, ROWS_PER_TILE)],
            agg_out.at[core, c // PAIR, pl.ds(rbase, ROWS_PER_TILE),
                       pl.ds((c % PAIR) * CW, CW)],
        )
        if with_deg and c == 0:
            pltpu.sync_copy(
                deg_sp.at[pl.ds(rbase, ROWS_PER_TILE)],
                deg_out.at[core, pl.ds(rbase, ROWS_PER_TILE)],
            )
        plsc.subcore_barrier()


def _sc_agg(C, with_deg, tab3, src, dst2d):
    mesh = plsc.VectorSubcoreMesh(core_axis_name="c", subcore_axis_name="s")
    out_type = [jax.ShapeDtypeStruct((NC, C // PAIR, N_PAD, PAIR * CW), jnp.float32)]
    if with_deg:
        out_type.append(jax.ShapeDtypeStruct((NC, N_PAD), jnp.float32))
    scratch = [
        pltpu.VMEM((EDGES_PER_W,), jnp.int32),
        pltpu.VMEM((CHUNKS_PER_W, EC), jnp.int32),
        pltpu.VMEM((EC, CW), jnp.float32),
        pltpu.VMEM((EC, CW), jnp.float32),
        pltpu.VMEM((ROWS_PER_TILE, CW), jnp.float32),
        pltpu.VMEM_SHARED((N_PAD, CW), jnp.float32),
    ]
    if with_deg:
        scratch += [
            pltpu.VMEM((ROWS_PER_TILE,), jnp.float32),
            pltpu.VMEM((EC,), jnp.float32),
            pltpu.VMEM_SHARED((N_PAD,), jnp.float32),
        ]
    scratch.append(pltpu.SemaphoreType.DMA)
    scratch.append(pltpu.SemaphoreType.DMA)
    fn = pl.kernel(
        functools.partial(_sc_agg_body, C, with_deg),
        out_type=out_type,
        mesh=mesh,
        scratch_types=scratch,
        compiler_params=pltpu.CompilerParams(use_tc_tiling_on_sc=False),
        name=f"sage_sc_agg_c{C}",
    )
    return fn(tab3, src, dst2d)


def _tc_dense_body(C, OUT_CHUNKS, relu, p_ref, x_ref, dt_ref, w_ref, b_ref, o_ref):
    dp = dt_ref[...]
    rdeg = 1.0 / (dp[:, 0:1] + dp[:, 1:2] + 1.0)
    acc = None
    for cp in range(C // PAIR):
        psum = p_ref[0, cp] + p_ref[1, cp]
        for h in range(PAIR):
            c = PAIR * cp + h
            hn = (psum[:, h * CW:(h + 1) * CW] + x_ref[c]) * rdeg
            part = jnp.dot(hn, w_ref[c], preferred_element_type=jnp.float32)
            acc = part if acc is None else acc + part
    out = acc + b_ref[0]
    if relu:
        out = jnp.maximum(out, 0.0)
    if OUT_CHUNKS is None:
        o_ref[...] = out
    else:
        for cc in range(OUT_CHUNKS):
            o_ref[cc] = out[:, cc * CW:(cc + 1) * CW]


def _tc_dense(C, relu, chunked_out, p, x_t, degt, w_r, b_r, out_dim):
    R = 1000
    grid = (N_NODES // R,)
    OUT_CHUNKS = out_dim // CW if chunked_out else None
    if chunked_out:
        out_shape = jax.ShapeDtypeStruct((out_dim // CW, N_NODES, CW), jnp.float32)
        out_spec = pl.BlockSpec((out_dim // CW, R, CW), lambda r: (0, r, 0))
    else:
        out_shape = jax.ShapeDtypeStruct((N_NODES, out_dim), jnp.float32)
        out_spec = pl.BlockSpec((R, out_dim), lambda r: (r, 0))
    return pl.pallas_call(
        functools.partial(_tc_dense_body, C, OUT_CHUNKS, relu),
        grid=grid,
        in_specs=[
            pl.BlockSpec((NC, C // PAIR, R, PAIR * CW), lambda r: (0, 0, r, 0)),
            pl.BlockSpec((C, R, CW), lambda r: (0, r, 0)),
            pl.BlockSpec((R, 2), lambda r: (r, 0)),
            pl.BlockSpec((C, CW, out_dim), lambda r: (0, 0, 0)),
            pl.BlockSpec((1, out_dim), lambda r: (0, 0)),
        ],
        out_specs=out_spec,
        out_shape=out_shape,
        name=f"sage_tc_dense_c{C}",
    )(p, x_t, degt, w_r, b_r)


def kernel(x, edge_index, W1, b1, W2, b2):
    src = edge_index[0].astype(jnp.int32)
    dst = edge_index[1].astype(jnp.int32)

    npad = E_PAD - N_EDGES
    pad_ids = jnp.arange(npad, dtype=jnp.int32)
    src_p = jnp.concatenate([src, pad_ids % N_NODES])
    dst_p = jnp.concatenate([dst, N_NODES + pad_ids % (N_PAD - N_NODES)])
    dst2d = dst_p.reshape(E_PAD // EC, EC)

    C1 = x.shape[1] // CW
    HID = W1.shape[1]
    C2 = HID // CW
    OUT = W2.shape[1]

    x_t = jnp.transpose(x.reshape(N_NODES, C1, CW), (1, 0, 2))
    w1_r = W1.reshape(C1, CW, HID)
    w2_r = W2.reshape(C2, CW, OUT)
    b1_r = b1.reshape(1, HID)
    b2_r = b2.reshape(1, OUT)

    p1, degp = _sc_agg(C1, True, x_t, src_p, dst2d)
    degt = jnp.transpose(degp, (1, 0))
    h1_t = _tc_dense(C1, True, True, p1, x_t, degt, w1_r, b1_r, HID)

    (p2,) = _sc_agg(C2, False, h1_t, src_p, dst2d)
    out = _tc_dense(C2, False, False, p2, h1_t, degt, w2_r, b2_r, OUT)
    return out


# speedup vs baseline: 1.0010x; 1.0010x over previous
import functools

import jax
import jax.numpy as jnp
from jax import lax
from jax.experimental import pallas as pl
from jax.experimental.pallas import tpu as pltpu
from jax.experimental.pallas import tpu_sc as plsc

N_NODES = 10000
N_EDGES = 160000
CW = 64
PAIR = 128 // CW
EC = 128

NC = 2
NS = 16
NW = NC * NS

N_PAD = 10240
ROWS_PER_TILE = N_PAD // NS
PIECES = ROWS_PER_TILE // EC
E_PAD = 163840
EDGES_PER_W = E_PAD // NW
CHUNKS_PER_W = EDGES_PER_W // EC


def _sc_agg_body(C, with_deg, *refs):
    tab3 = refs[0]
    src_hbm = refs[1]
    dst_hbm = refs[2]
    pos = 3
    agg_out = refs[pos]; pos += 1
    if with_deg:
        deg_out = refs[pos]; pos += 1
    src_v = refs[pos]; pos += 1
    dst_v = refs[pos]; pos += 1
    gbuf0 = refs[pos]; pos += 1
    gbuf1 = refs[pos]; pos += 1
    zbuf = refs[pos]; pos += 1
    acc_sp = refs[pos]; pos += 1
    if with_deg:
        degbuf = refs[pos]; pos += 1
        ones_v = refs[pos]; pos += 1
        deg_sp = refs[pos]; pos += 1
    gsem0 = refs[pos]; pos += 1
    gsem1 = refs[pos]; pos += 1

    core = lax.axis_index("c")
    sub = lax.axis_index("s")
    wid = sub * NC + core
    ebase = wid * EDGES_PER_W
    rbase = sub * ROWS_PER_TILE

    pltpu.sync_copy(src_hbm.at[pl.ds(ebase, EDGES_PER_W)], src_v)
    pltpu.sync_copy(dst_hbm.at[pl.ds(wid * CHUNKS_PER_W, CHUNKS_PER_W)], dst_v)

    z16 = jnp.zeros((16,), jnp.float32)

    def zrow(r, carry):
        for k in range(CW // 16):
            zbuf[r, pl.ds(k * 16, 16)] = z16
        return carry

    lax.fori_loop(0, ROWS_PER_TILE, zrow, 0)
    if with_deg:
        one16 = jnp.ones((16,), jnp.float32)

        def dz(i, carry):
            degbuf[pl.ds(i * 16, 16)] = z16
            return carry

        lax.fori_loop(0, ROWS_PER_TILE // 16, dz, 0)
        for k in range(EC // 16):
            ones_v[pl.ds(k * 16, 16)] = one16

    for c in range(C):
        pltpu.sync_copy(zbuf, acc_sp.at[pl.ds(rbase, ROWS_PER_TILE)])
        if with_deg and c == 0:
            pltpu.sync_copy(degbuf, deg_sp.at[pl.ds(rbase, ROWS_PER_TILE)])
        plsc.subcore_barrier()

        tab = tab3.at[c]

        def start_gather(j, buf, sem):
            pltpu.async_copy(tab.at[src_v.at[pl.ds(j * EC, EC)]], buf, sem)

        def wait_gather(buf, sem):
            pltpu.make_async_copy(tab.at[pl.ds(0, EC)], buf, sem).wait()

        def scatter(j, buf):
            pltpu.sync_copy(buf, acc_sp.at[dst_v.at[j]], add=True)
            if with_deg and c == 0:
                pltpu.sync_copy(ones_v, deg_sp.at[dst_v.at[j]], add=True)

        start_gather(0, gbuf0, gsem0)

        def pair(k, carry):
            j0 = 2 * k
            start_gather(j0 + 1, gbuf1, gsem1)
            wait_gather(gbuf0, gsem0)
            scatter(j0, gbuf0)
            start_gather(j0 + 2, gbuf0, gsem0)
            wait_gather(gbuf1, gsem1)
            scatter(j0 + 1, gbuf1)
            return carry

        lax.fori_loop(0, CHUNKS_PER_W // 2 - 1, pair, 0)
        start_gather(CHUNKS_PER_W - 1, gbuf1, gsem1)
        wait_gather(gbuf0, gsem0)
        scatter(CHUNKS_PER_W - 2, gbuf0)
        wait_gather(gbuf1, gsem1)
        scatter(CHUNKS_PER_W - 1, gbuf1)
        plsc.subcore_barrier()

        pltpu.sync_copy(
            acc_sp.at[pl.ds(rbase, ROWS_PER_TILE)],
            agg_out.at[core, c // PAIR, pl.ds(rbase, ROWS_PER_TILE),
                       pl.ds((c % PAIR) * CW, CW)],
        )
        if with_deg and c == 0:
            pltpu.sync_copy(
                deg_sp.at[pl.ds(rbase, ROWS_PER_TILE)],
                deg_out.at[core, pl.ds(rbase, ROWS_PER_TILE)],
            )
        plsc.subcore_barrier()


def _sc_agg(C, with_deg, tab3, src, dst2d):
    mesh = plsc.VectorSubcoreMesh(core_axis_name="c", subcore_axis_name="s")
    out_type = [jax.ShapeDtypeStruct((NC, C // PAIR, N_PAD, PAIR * CW), jnp.float32)]
    if with_deg:
        out_type.append(jax.ShapeDtypeStruct((NC, N_PAD), jnp.float32))
    scratch = [
        pltpu.VMEM((EDGES_PER_W,), jnp.int32),
        pltpu.VMEM((CHUNKS_PER_W, EC), jnp.int32),
        pltpu.VMEM((EC, CW), jnp.float32),
        pltpu.VMEM((EC, CW), jnp.float32),
        pltpu.VMEM((ROWS_PER_TILE, CW), jnp.float32),
        pltpu.VMEM_SHARED((N_PAD, CW), jnp.float32),
    ]
    if with_deg:
        scratch += [
            pltpu.VMEM((ROWS_PER_TILE,), jnp.float32),
            pltpu.VMEM((EC,), jnp.float32),
            pltpu.VMEM_SHARED((N_PAD,), jnp.float32),
        ]
    scratch.append(pltpu.SemaphoreType.DMA)
    scratch.append(pltpu.SemaphoreType.DMA)
    fn = pl.kernel(
        functools.partial(_sc_agg_body, C, with_deg),
        out_type=out_type,
        mesh=mesh,
        scratch_types=scratch,
        compiler_params=pltpu.CompilerParams(use_tc_tiling_on_sc=False),
        name=f"sage_sc_agg_c{C}",
    )
    return fn(tab3, src, dst2d)


def _tc_dense_body(C, OUT_CHUNKS, relu, has_acc, *refs):
    if has_acc:
        p_ref, x_ref, dt_ref, w_ref, b_ref, a_ref, o_ref = refs
    else:
        p_ref, x_ref, dt_ref, w_ref, b_ref, o_ref = refs
    dp = dt_ref[...]
    rdeg = 1.0 / (dp[:, 0:1] + dp[:, 1:2] + 1.0)
    acc = a_ref[...] if has_acc else None
    for cp in range(C // PAIR):
        psum = p_ref[0, cp] + p_ref[1, cp]
        for h in range(PAIR):
            c = PAIR * cp + h
            hn = (psum[:, h * CW:(h + 1) * CW] + x_ref[c]) * rdeg
            part = jnp.dot(hn, w_ref[c], preferred_element_type=jnp.float32)
            acc = part if acc is None else acc + part
    out = acc + b_ref[0]
    if relu:
        out = jnp.maximum(out, 0.0)
    if OUT_CHUNKS is None:
        o_ref[...] = out
    else:
        for cc in range(OUT_CHUNKS):
            o_ref[cc] = out[:, cc * CW:(cc + 1) * CW]


def _tc_dense(C, relu, chunked_out, p, x_t, degt, w_r, b_r, out_dim, acc_in=None):
    R = 1000
    grid = (N_NODES // R,)
    OUT_CHUNKS = out_dim // CW if chunked_out else None
    if chunked_out:
        out_shape = jax.ShapeDtypeStruct((out_dim // CW, N_NODES, CW), jnp.float32)
        out_spec = pl.BlockSpec((out_dim // CW, R, CW), lambda r: (0, r, 0))
    else:
        out_shape = jax.ShapeDtypeStruct((N_NODES, out_dim), jnp.float32)
        out_spec = pl.BlockSpec((R, out_dim), lambda r: (r, 0))
    in_specs = [
        pl.BlockSpec((NC, C // PAIR, R, PAIR * CW), lambda r: (0, 0, r, 0)),
        pl.BlockSpec((C, R, CW), lambda r: (0, r, 0)),
        pl.BlockSpec((R, 2), lambda r: (r, 0)),
        pl.BlockSpec((C, CW, out_dim), lambda r: (0, 0, 0)),
        pl.BlockSpec((1, out_dim), lambda r: (0, 0)),
    ]
    args = [p, x_t, degt, w_r, b_r]
    if acc_in is not None:
        in_specs.append(pl.BlockSpec((R, out_dim), lambda r: (r, 0)))
        args.append(acc_in)
    return pl.pallas_call(
        functools.partial(_tc_dense_body, C, OUT_CHUNKS, relu, acc_in is not None),
        grid=grid,
        in_specs=in_specs,
        out_specs=out_spec,
        out_shape=out_shape,
        name=f"sage_tc_dense_c{C}",
    )(*args)


def kernel(x, edge_index, W1, b1, W2, b2):
    src = edge_index[0].astype(jnp.int32)
    dst = edge_index[1].astype(jnp.int32)

    npad = E_PAD - N_EDGES
    pad_ids = jnp.arange(npad, dtype=jnp.int32)
    src_p = jnp.concatenate([src, pad_ids % N_NODES])
    dst_p = jnp.concatenate([dst, N_NODES + pad_ids % (N_PAD - N_NODES)])
    dst2d = dst_p.reshape(E_PAD // EC, EC)

    C1 = x.shape[1] // CW
    HID = W1.shape[1]
    C2 = HID // CW
    OUT = W2.shape[1]

    x_t = jnp.transpose(x.reshape(N_NODES, C1, CW), (1, 0, 2))
    w1_r = W1.reshape(C1, CW, HID)
    w2_r = W2.reshape(C2, CW, OUT)
    b1_r = b1.reshape(1, HID)
    b2_r = b2.reshape(1, OUT)

    p1, degp = _sc_agg(C1, True, x_t, src_p, dst2d)
    degt = jnp.transpose(degp, (1, 0))
    Hh = HID // 2
    C2h = C2 // 2
    h1a = _tc_dense(C1, True, True, p1, x_t, degt,
                    w1_r[:, :, :Hh], b1_r[:, :Hh], Hh)
    h1b = _tc_dense(C1, True, True, p1, x_t, degt,
                    w1_r[:, :, Hh:], b1_r[:, Hh:], Hh)

    (p2a,) = _sc_agg(C2h, False, h1a, src_p, dst2d)
    (p2b,) = _sc_agg(C2h, False, h1b, src_p, dst2d)
    part = _tc_dense(C2h, False, False, p2a, h1a, degt,
                     w2_r[:C2h], b2_r, OUT)
    out = _tc_dense(C2h, False, False, p2b, h1b, degt,
                    w2_r[C2h:], jnp.zeros_like(b2_r), OUT, acc_in=part)
    return out


# speedup vs baseline: 1.0466x; 1.0455x over previous
import functools

import jax
import jax.numpy as jnp
from jax import lax
from jax.experimental import pallas as pl
from jax.experimental.pallas import tpu as pltpu
from jax.experimental.pallas import tpu_sc as plsc

N_NODES = 10000
N_EDGES = 160000
CW = 64
PAIR = 128 // CW
EC = 128

NC = 2
NS = 16
NW = NC * NS

N_PAD = 10240
ROWS_PER_TILE = N_PAD // NS
PIECES = ROWS_PER_TILE // EC
E_PAD = 163840
EDGES_PER_W = E_PAD // NW
CHUNKS_PER_W = EDGES_PER_W // EC


def _sc_agg_body(C, with_deg, *refs):
    tab3 = refs[0]
    src_hbm = refs[1]
    dst_hbm = refs[2]
    pos = 3
    agg_out = refs[pos]; pos += 1
    if with_deg:
        deg_out = refs[pos]; pos += 1
    src_v = refs[pos]; pos += 1
    dst_v = refs[pos]; pos += 1
    bufs = refs[pos:pos + 4]; pos += 4
    zbuf = refs[pos]; pos += 1
    acc_sp = refs[pos]; pos += 1
    if with_deg:
        degbuf = refs[pos]; pos += 1
        ones_v = refs[pos]; pos += 1
        deg_sp = refs[pos]; pos += 1
    gsems = refs[pos:pos + 4]; pos += 4
    ssems = refs[pos:pos + 4]; pos += 4

    core = lax.axis_index("c")
    sub = lax.axis_index("s")
    wid = sub * NC + core
    ebase = wid * EDGES_PER_W
    rbase = sub * ROWS_PER_TILE

    pltpu.sync_copy(src_hbm.at[pl.ds(ebase, EDGES_PER_W)], src_v)
    pltpu.sync_copy(dst_hbm.at[pl.ds(wid * CHUNKS_PER_W, CHUNKS_PER_W)], dst_v)

    z16 = jnp.zeros((16,), jnp.float32)

    def zrow(r, carry):
        for k in range(CW // 16):
            zbuf[r, pl.ds(k * 16, 16)] = z16
        return carry

    lax.fori_loop(0, ROWS_PER_TILE, zrow, 0)
    if with_deg:
        one16 = jnp.ones((16,), jnp.float32)

        def dz(i, carry):
            degbuf[pl.ds(i * 16, 16)] = z16
            return carry

        lax.fori_loop(0, ROWS_PER_TILE // 16, dz, 0)
        for k in range(EC // 16):
            ones_v[pl.ds(k * 16, 16)] = one16

    for c in range(C):
        pltpu.sync_copy(zbuf, acc_sp.at[pl.ds(rbase, ROWS_PER_TILE)])
        if with_deg and c == 0:
            pltpu.sync_copy(degbuf, deg_sp.at[pl.ds(rbase, ROWS_PER_TILE)])
        plsc.subcore_barrier()

        tab = tab3.at[c]

        def start_gather(j, i):
            pltpu.async_copy(tab.at[src_v.at[pl.ds(j * EC, EC)]], bufs[i], gsems[i])

        def wait_gather(i):
            pltpu.make_async_copy(tab.at[pl.ds(0, EC)], bufs[i], gsems[i]).wait()

        def scatter_async(j, i):
            pltpu.async_copy(bufs[i], acc_sp.at[dst_v.at[j]], ssems[i], add=True)
            if with_deg and c == 0:
                pltpu.sync_copy(ones_v, deg_sp.at[dst_v.at[j]], add=True)

        def wait_scatter(i):
            pltpu.make_async_copy(
                bufs[i], acc_sp.at[dst_v.at[0]], ssems[i]
            ).wait()

        def step(j, i, pre_j, pre_i, drain):
            wait_gather(i)
            scatter_async(j, i)
            if drain:
                wait_scatter(pre_i)
            if pre_j is not None:
                start_gather(pre_j, pre_i)

        J = CHUNKS_PER_W
        start_gather(0, 0)
        start_gather(1, 1)
        step(0, 0, 2, 2, False)
        step(1, 1, 3, 3, False)
        step(2, 2, 4, 0, True)
        step(3, 3, 5, 1, True)

        def quad(q, carry):
            j0 = 4 * q
            for t in range(4):
                step(j0 + t, t, j0 + t + 2, (t + 2) % 4, True)
            return carry

        lax.fori_loop(1, J // 4 - 1, quad, 0)
        step(J - 4, 0, J - 2, 2, True)
        step(J - 3, 1, J - 1, 3, True)
        step(J - 2, 2, None, None, False)
        step(J - 1, 3, None, None, False)
        for i in range(4):
            wait_scatter(i)
        plsc.subcore_barrier()

        pltpu.sync_copy(
            acc_sp.at[pl.ds(rbase, ROWS_PER_TILE)],
            agg_out.at[core, c // PAIR, pl.ds(rbase, ROWS_PER_TILE),
                       pl.ds((c % PAIR) * CW, CW)],
        )
        if with_deg and c == 0:
            pltpu.sync_copy(
                deg_sp.at[pl.ds(rbase, ROWS_PER_TILE)],
                deg_out.at[core, pl.ds(rbase, ROWS_PER_TILE)],
            )
        plsc.subcore_barrier()


def _sc_agg(C, with_deg, tab3, src, dst2d):
    mesh = plsc.VectorSubcoreMesh(core_axis_name="c", subcore_axis_name="s")
    out_type = [jax.ShapeDtypeStruct((NC, C // PAIR, N_PAD, PAIR * CW), jnp.float32)]
    if with_deg:
        out_type.append(jax.ShapeDtypeStruct((NC, N_PAD), jnp.float32))
    scratch = [
        pltpu.VMEM((EDGES_PER_W,), jnp.int32),
        pltpu.VMEM((CHUNKS_PER_W, EC), jnp.int32),
        pltpu.VMEM((EC, CW), jnp.float32),
        pltpu.VMEM((EC, CW), jnp.float32),
        pltpu.VMEM((EC, CW), jnp.float32),
        pltpu.VMEM((EC, CW), jnp.float32),
        pltpu.VMEM((ROWS_PER_TILE, CW), jnp.float32),
        pltpu.VMEM_SHARED((N_PAD, CW), jnp.float32),
    ]
    if with_deg:
        scratch += [
            pltpu.VMEM((ROWS_PER_TILE,), jnp.float32),
            pltpu.VMEM((EC,), jnp.float32),
            pltpu.VMEM_SHARED((N_PAD,), jnp.float32),
        ]
    scratch += [pltpu.SemaphoreType.DMA] * 8
    fn = pl.kernel(
        functools.partial(_sc_agg_body, C, with_deg),
        out_type=out_type,
        mesh=mesh,
        scratch_types=scratch,
        compiler_params=pltpu.CompilerParams(use_tc_tiling_on_sc=False),
        name=f"sage_sc_agg_c{C}",
    )
    return fn(tab3, src, dst2d)


def _tc_dense_body(C, OUT_CHUNKS, relu, has_acc, *refs):
    if has_acc:
        p_ref, x_ref, dt_ref, w_ref, b_ref, a_ref, o_ref = refs
    else:
        p_ref, x_ref, dt_ref, w_ref, b_ref, o_ref = refs
    dp = dt_ref[...]
    rdeg = 1.0 / (dp[:, 0:1] + dp[:, 1:2] + 1.0)
    acc = a_ref[...] if has_acc else None
    for cp in range(C // PAIR):
        psum = p_ref[0, cp] + p_ref[1, cp]
        for h in range(PAIR):
            c = PAIR * cp + h
            hn = (psum[:, h * CW:(h + 1) * CW] + x_ref[c]) * rdeg
            part = jnp.dot(hn, w_ref[c], preferred_element_type=jnp.float32)
            acc = part if acc is None else acc + part
    out = acc + b_ref[0]
    if relu:
        out = jnp.maximum(out, 0.0)
    if OUT_CHUNKS is None:
        o_ref[...] = out
    else:
        for cc in range(OUT_CHUNKS):
            o_ref[cc] = out[:, cc * CW:(cc + 1) * CW]


def _tc_dense(C, relu, chunked_out, p, x_t, degt, w_r, b_r, out_dim, acc_in=None):
    R = 1000
    grid = (N_NODES // R,)
    OUT_CHUNKS = out_dim // CW if chunked_out else None
    if chunked_out:
        out_shape = jax.ShapeDtypeStruct((out_dim // CW, N_NODES, CW), jnp.float32)
        out_spec = pl.BlockSpec((out_dim // CW, R, CW), lambda r: (0, r, 0))
    else:
        out_shape = jax.ShapeDtypeStruct((N_NODES, out_dim), jnp.float32)
        out_spec = pl.BlockSpec((R, out_dim), lambda r: (r, 0))
    in_specs = [
        pl.BlockSpec((NC, C // PAIR, R, PAIR * CW), lambda r: (0, 0, r, 0)),
        pl.BlockSpec((C, R, CW), lambda r: (0, r, 0)),
        pl.BlockSpec((R, 2), lambda r: (r, 0)),
        pl.BlockSpec((C, CW, out_dim), lambda r: (0, 0, 0)),
        pl.BlockSpec((1, out_dim), lambda r: (0, 0)),
    ]
    args = [p, x_t, degt, w_r, b_r]
    if acc_in is not None:
        in_specs.append(pl.BlockSpec((R, out_dim), lambda r: (r, 0)))
        args.append(acc_in)
    return pl.pallas_call(
        functools.partial(_tc_dense_body, C, OUT_CHUNKS, relu, acc_in is not None),
        grid=grid,
        in_specs=in_specs,
        out_specs=out_spec,
        out_shape=out_shape,
        name=f"sage_tc_dense_c{C}",
    )(*args)


def kernel(x, edge_index, W1, b1, W2, b2):
    src = edge_index[0].astype(jnp.int32)
    dst = edge_index[1].astype(jnp.int32)

    npad = E_PAD - N_EDGES
    pad_ids = jnp.arange(npad, dtype=jnp.int32)
    src_p = jnp.concatenate([src, pad_ids % N_NODES])
    dst_p = jnp.concatenate([dst, N_NODES + pad_ids % (N_PAD - N_NODES)])
    dst2d = dst_p.reshape(E_PAD // EC, EC)

    C1 = x.shape[1] // CW
    HID = W1.shape[1]
    C2 = HID // CW
    OUT = W2.shape[1]

    x_t = jnp.transpose(x.reshape(N_NODES, C1, CW), (1, 0, 2))
    w1_r = W1.reshape(C1, CW, HID)
    w2_r = W2.reshape(C2, CW, OUT)
    b1_r = b1.reshape(1, HID)
    b2_r = b2.reshape(1, OUT)

    p1, degp = _sc_agg(C1, True, x_t, src_p, dst2d)
    degt = jnp.transpose(degp, (1, 0))
    Hh = HID // 2
    C2h = C2 // 2
    h1a = _tc_dense(C1, True, True, p1, x_t, degt,
                    w1_r[:, :, :Hh], b1_r[:, :Hh], Hh)
    h1b = _tc_dense(C1, True, True, p1, x_t, degt,
                    w1_r[:, :, Hh:], b1_r[:, Hh:], Hh)

    (p2a,) = _sc_agg(C2h, False, h1a, src_p, dst2d)
    (p2b,) = _sc_agg(C2h, False, h1b, src_p, dst2d)
    part = _tc_dense(C2h, False, False, p2a, h1a, degt,
                     w2_r[:C2h], b2_r, OUT)
    out = _tc_dense(C2h, False, False, p2b, h1b, degt,
                    w2_r[C2h:], jnp.zeros_like(b2_r), OUT, acc_in=part)
    return out


# speedup vs baseline: 1.0990x; 1.0500x over previous
import functools

import jax
import jax.numpy as jnp
from jax import lax
from jax.experimental import pallas as pl
from jax.experimental.pallas import tpu as pltpu
from jax.experimental.pallas import tpu_sc as plsc

N_NODES = 10000
N_EDGES = 160000
CW = 64
PAIR = 128 // CW
EC = 256

NC = 2
NS = 16
NW = NC * NS

N_PAD = 10240
ROWS_PER_TILE = N_PAD // NS
PIECES = ROWS_PER_TILE // EC
E_PAD = 163840
EDGES_PER_W = E_PAD // NW
CHUNKS_PER_W = EDGES_PER_W // EC


def _sc_agg_body(C, with_deg, *refs):
    tab3 = refs[0]
    src_hbm = refs[1]
    dst_hbm = refs[2]
    pos = 3
    agg_out = refs[pos]; pos += 1
    if with_deg:
        deg_out = refs[pos]; pos += 1
    src_v = refs[pos]; pos += 1
    dst_v = refs[pos]; pos += 1
    bufs = refs[pos:pos + 4]; pos += 4
    zbuf = refs[pos]; pos += 1
    acc_sp = refs[pos]; pos += 1
    if with_deg:
        degbuf = refs[pos]; pos += 1
        ones_v = refs[pos]; pos += 1
        deg_sp = refs[pos]; pos += 1
    gsems = refs[pos:pos + 4]; pos += 4
    ssems = refs[pos:pos + 4]; pos += 4

    core = lax.axis_index("c")
    sub = lax.axis_index("s")
    wid = sub * NC + core
    ebase = wid * EDGES_PER_W
    rbase = sub * ROWS_PER_TILE

    pltpu.sync_copy(src_hbm.at[pl.ds(ebase, EDGES_PER_W)], src_v)
    pltpu.sync_copy(dst_hbm.at[pl.ds(wid * CHUNKS_PER_W, CHUNKS_PER_W)], dst_v)

    z16 = jnp.zeros((16,), jnp.float32)

    def zrow(r, carry):
        for k in range(CW // 16):
            zbuf[r, pl.ds(k * 16, 16)] = z16
        return carry

    lax.fori_loop(0, 128, zrow, 0)
    if with_deg:
        one16 = jnp.ones((16,), jnp.float32)

        def dz(i, carry):
            degbuf[pl.ds(i * 16, 16)] = z16
            return carry

        lax.fori_loop(0, ROWS_PER_TILE // 16, dz, 0)
        for k in range(EC // 16):
            ones_v[pl.ds(k * 16, 16)] = one16

    for c in range(C):
        for p in range(ROWS_PER_TILE // 128):
            pltpu.sync_copy(zbuf, acc_sp.at[pl.ds(rbase + p * 128, 128)])
        if with_deg and c == 0:
            pltpu.sync_copy(degbuf, deg_sp.at[pl.ds(rbase, ROWS_PER_TILE)])
        plsc.subcore_barrier()

        tab = tab3.at[c]

        def start_gather(j, i):
            pltpu.async_copy(tab.at[src_v.at[pl.ds(j * EC, EC)]], bufs[i], gsems[i])

        def wait_gather(i):
            pltpu.make_async_copy(tab.at[pl.ds(0, EC)], bufs[i], gsems[i]).wait()

        def scatter_async(j, i):
            pltpu.async_copy(bufs[i], acc_sp.at[dst_v.at[j]], ssems[i], add=True)
            if with_deg and c == 0:
                pltpu.sync_copy(ones_v, deg_sp.at[dst_v.at[j]], add=True)

        def wait_scatter(i):
            pltpu.make_async_copy(
                bufs[i], acc_sp.at[dst_v.at[0]], ssems[i]
            ).wait()

        def step(j, i, pre_j, pre_i, drain):
            wait_gather(i)
            scatter_async(j, i)
            if drain:
                wait_scatter(pre_i)
            if pre_j is not None:
                start_gather(pre_j, pre_i)

        J = CHUNKS_PER_W
        start_gather(0, 0)
        start_gather(1, 1)
        step(0, 0, 2, 2, False)
        step(1, 1, 3, 3, False)
        step(2, 2, 4, 0, True)
        step(3, 3, 5, 1, True)

        def quad(q, carry):
            j0 = 4 * q
            for t in range(4):
                step(j0 + t, t, j0 + t + 2, (t + 2) % 4, True)
            return carry

        lax.fori_loop(1, J // 4 - 1, quad, 0)
        step(J - 4, 0, J - 2, 2, True)
        step(J - 3, 1, J - 1, 3, True)
        step(J - 2, 2, None, None, False)
        step(J - 1, 3, None, None, False)
        for i in range(4):
            wait_scatter(i)
        plsc.subcore_barrier()

        pltpu.sync_copy(
            acc_sp.at[pl.ds(rbase, ROWS_PER_TILE)],
            agg_out.at[core, c // PAIR, pl.ds(rbase, ROWS_PER_TILE),
                       pl.ds((c % PAIR) * CW, CW)],
        )
        if with_deg and c == 0:
            pltpu.sync_copy(
                deg_sp.at[pl.ds(rbase, ROWS_PER_TILE)],
                deg_out.at[core, pl.ds(rbase, ROWS_PER_TILE)],
            )
        plsc.subcore_barrier()


def _sc_agg(C, with_deg, tab3, src, dst2d):
    mesh = plsc.VectorSubcoreMesh(core_axis_name="c", subcore_axis_name="s")
    out_type = [jax.ShapeDtypeStruct((NC, C // PAIR, N_PAD, PAIR * CW), jnp.float32)]
    if with_deg:
        out_type.append(jax.ShapeDtypeStruct((NC, N_PAD), jnp.float32))
    scratch = [
        pltpu.VMEM((EDGES_PER_W,), jnp.int32),
        pltpu.VMEM((CHUNKS_PER_W, EC), jnp.int32),
        pltpu.VMEM((EC, CW), jnp.float32),
        pltpu.VMEM((EC, CW), jnp.float32),
        pltpu.VMEM((EC, CW), jnp.float32),
        pltpu.VMEM((EC, CW), jnp.float32),
        pltpu.VMEM((128, CW), jnp.float32),
        pltpu.VMEM_SHARED((N_PAD, CW), jnp.float32),
    ]
    if with_deg:
        scratch += [
            pltpu.VMEM((ROWS_PER_TILE,), jnp.float32),
            pltpu.VMEM((EC,), jnp.float32),
            pltpu.VMEM_SHARED((N_PAD,), jnp.float32),
        ]
    scratch += [pltpu.SemaphoreType.DMA] * 8
    fn = pl.kernel(
        functools.partial(_sc_agg_body, C, with_deg),
        out_type=out_type,
        mesh=mesh,
        scratch_types=scratch,
        compiler_params=pltpu.CompilerParams(use_tc_tiling_on_sc=False),
        name=f"sage_sc_agg_c{C}",
    )
    return fn(tab3, src, dst2d)


def _tc_dense_body(C, OUT_CHUNKS, relu, has_acc, *refs):
    if has_acc:
        p_ref, x_ref, dt_ref, w_ref, b_ref, a_ref, o_ref = refs
    else:
        p_ref, x_ref, dt_ref, w_ref, b_ref, o_ref = refs
    dp = dt_ref[...]
    rdeg = 1.0 / (dp[:, 0:1] + dp[:, 1:2] + 1.0)
    acc = a_ref[...] if has_acc else None
    for cp in range(C // PAIR):
        psum = p_ref[0, cp] + p_ref[1, cp]
        for h in range(PAIR):
            c = PAIR * cp + h
            hn = (psum[:, h * CW:(h + 1) * CW] + x_ref[c]) * rdeg
            part = jnp.dot(hn, w_ref[c], preferred_element_type=jnp.float32)
            acc = part if acc is None else acc + part
    out = acc + b_ref[0]
    if relu:
        out = jnp.maximum(out, 0.0)
    if OUT_CHUNKS is None:
        o_ref[...] = out
    else:
        for cc in range(OUT_CHUNKS):
            o_ref[cc] = out[:, cc * CW:(cc + 1) * CW]


def _tc_dense(C, relu, chunked_out, p, x_t, degt, w_r, b_r, out_dim, acc_in=None):
    R = 1000
    grid = (N_NODES // R,)
    OUT_CHUNKS = out_dim // CW if chunked_out else None
    if chunked_out:
        out_shape = jax.ShapeDtypeStruct((out_dim // CW, N_NODES, CW), jnp.float32)
        out_spec = pl.BlockSpec((out_dim // CW, R, CW), lambda r: (0, r, 0))
    else:
        out_shape = jax.ShapeDtypeStruct((N_NODES, out_dim), jnp.float32)
        out_spec = pl.BlockSpec((R, out_dim), lambda r: (r, 0))
    in_specs = [
        pl.BlockSpec((NC, C // PAIR, R, PAIR * CW), lambda r: (0, 0, r, 0)),
        pl.BlockSpec((C, R, CW), lambda r: (0, r, 0)),
        pl.BlockSpec((R, 2), lambda r: (r, 0)),
        pl.BlockSpec((C, CW, out_dim), lambda r: (0, 0, 0)),
        pl.BlockSpec((1, out_dim), lambda r: (0, 0)),
    ]
    args = [p, x_t, degt, w_r, b_r]
    if acc_in is not None:
        in_specs.append(pl.BlockSpec((R, out_dim), lambda r: (r, 0)))
        args.append(acc_in)
    return pl.pallas_call(
        functools.partial(_tc_dense_body, C, OUT_CHUNKS, relu, acc_in is not None),
        grid=grid,
        in_specs=in_specs,
        out_specs=out_spec,
        out_shape=out_shape,
        name=f"sage_tc_dense_c{C}",
    )(*args)


def kernel(x, edge_index, W1, b1, W2, b2):
    src = edge_index[0].astype(jnp.int32)
    dst = edge_index[1].astype(jnp.int32)

    npad = E_PAD - N_EDGES
    pad_ids = jnp.arange(npad, dtype=jnp.int32)
    src_p = jnp.concatenate([src, pad_ids % N_NODES])
    dst_p = jnp.concatenate([dst, N_NODES + pad_ids % (N_PAD - N_NODES)])
    dst2d = dst_p.reshape(E_PAD // EC, EC)

    C1 = x.shape[1] // CW
    HID = W1.shape[1]
    C2 = HID // CW
    OUT = W2.shape[1]

    x_t = jnp.transpose(x.reshape(N_NODES, C1, CW), (1, 0, 2))
    w1_r = W1.reshape(C1, CW, HID)
    w2_r = W2.reshape(C2, CW, OUT)
    b1_r = b1.reshape(1, HID)
    b2_r = b2.reshape(1, OUT)

    p1, degp = _sc_agg(C1, True, x_t, src_p, dst2d)
    degt = jnp.transpose(degp, (1, 0))
    Hh = HID // 2
    C2h = C2 // 2
    h1a = _tc_dense(C1, True, True, p1, x_t, degt,
                    w1_r[:, :, :Hh], b1_r[:, :Hh], Hh)
    h1b = _tc_dense(C1, True, True, p1, x_t, degt,
                    w1_r[:, :, Hh:], b1_r[:, Hh:], Hh)

    (p2a,) = _sc_agg(C2h, False, h1a, src_p, dst2d)
    (p2b,) = _sc_agg(C2h, False, h1b, src_p, dst2d)
    part = _tc_dense(C2h, False, False, p2a, h1a, degt,
                     w2_r[:C2h], b2_r, OUT)
    out = _tc_dense(C2h, False, False, p2b, h1b, degt,
                    w2_r[C2h:], jnp.zeros_like(b2_r), OUT, acc_in=part)
    return out
